# Initial kernel scaffold; baseline (speedup 1.0000x reference)
import functools

import jax
import jax.numpy as jnp
from jax import lax
from jax.experimental import pallas as pl
from jax.experimental.pallas import tpu as pltpu
from jax.experimental.pallas import tpu_sc as plsc

N = 10000
E = 160000
D = 256
HALF = 128
OUT = 128

NUM_TILES = 16
EDGES_PER_TILE = E // NUM_TILES
KB = 400
NB = EDGES_PER_TILE // KB
ROWS_PER_TILE = N // NUM_TILES
CW = 8


def _sc_segment_sum(count: bool):
    mesh = plsc.VectorSubcoreMesh(core_axis_name="c", subcore_axis_name="s")
    out_type = [
        jax.ShapeDtypeStruct((N, HALF), jnp.float32),
        jax.ShapeDtypeStruct((N, HALF), jnp.float32),
    ]
    if count:
        out_type.append(jax.ShapeDtypeStruct((N, CW), jnp.float32))

    scratch = [
        pltpu.VMEM_SHARED((N, HALF), jnp.float32),
        pltpu.VMEM((NB, KB), jnp.int32),
        pltpu.VMEM((NB, KB), jnp.int32),
        pltpu.VMEM((KB, HALF), jnp.float32),
        pltpu.SemaphoreType.DMA,
    ]
    if count:
        scratch.insert(1, pltpu.VMEM_SHARED((N, CW), jnp.float32))
        scratch.append(pltpu.VMEM((KB, CW), jnp.float32))

    @functools.partial(
        pl.kernel, mesh=mesh, out_type=tuple(out_type),
        scratch_types=scratch,
    )
    def k(*refs):
        if count:
            (tabL, tabR, srcr, dstr, ones_hbm, zrow, zcnt,
             sumL, sumR, cnt_out,
             acc, cacc, idx_s, idx_d, rows, sem, ones_v) = refs
        else:
            (tabL, tabR, srcr, dstr, zrow,
             sumL, sumR,
             acc, idx_s, idx_d, rows, sem) = refs

        c = lax.axis_index("c")
        s = lax.axis_index("s")
        row0 = s * ROWS_PER_TILE

        pltpu.sync_copy(zrow, acc.at[pl.ds(row0, ROWS_PER_TILE)])
        pltpu.sync_copy(srcr.at[s], idx_s)
        pltpu.sync_copy(dstr.at[s], idx_d)

        def run(table, out, with_count):
            if with_count:
                pltpu.sync_copy(zcnt, cacc.at[pl.ds(row0, ROWS_PER_TILE)])
                pltpu.sync_copy(ones_hbm, ones_v)
            plsc.subcore_barrier()
            for j in range(NB):
                pltpu.async_copy(table.at[idx_s.at[j]], rows, sem).wait()
                pltpu.sync_copy(rows, acc.at[idx_d.at[j]], add=True)
                if with_count:
                    pltpu.sync_copy(ones_v, cacc.at[idx_d.at[j]], add=True)
            plsc.subcore_barrier()
            pltpu.sync_copy(acc.at[pl.ds(row0, ROWS_PER_TILE)],
                            out.at[pl.ds(row0, ROWS_PER_TILE)])
            if with_count:
                pltpu.sync_copy(cacc.at[pl.ds(row0, ROWS_PER_TILE)],
                                cnt_out.at[pl.ds(row0, ROWS_PER_TILE)])

        @pl.when(c == 0)
        def _():
            run(tabL, sumL, count)

        @pl.when(c == 1)
        def _():
            run(tabR, sumR, False)

    return k


_sc_seg_count = _sc_segment_sum(True)
_sc_seg = _sc_segment_sum(False)

R = 1000


def _tc_layer_body(sL, sR, hL, hR, cnt, Wl, Wr, b, oL, oR):
    inv = 1.0 / jnp.maximum(cnt[:, 0:1], 1.0)
    acc = (jnp.dot(sL[...] * inv, Wl[0], precision="highest")
           + jnp.dot(sR[...] * inv, Wl[1], precision="highest")
           + jnp.dot(hL[...], Wr[0], precision="highest")
           + jnp.dot(hR[...], Wr[1], precision="highest")
           + b[...])
    acc = jnp.maximum(acc, 0.0)
    oL[...] = acc[:, :HALF]
    oR[...] = acc[:, HALF:]


def _tc_layer(sL, sR, hL, hR, cnt, Wl, Wr, b):
    grid = (N // R,)
    row_spec = pl.BlockSpec((R, HALF), lambda i: (i, 0))
    return pl.pallas_call(
        _tc_layer_body,
        grid=grid,
        in_specs=[
            row_spec, row_spec, row_spec, row_spec,
            pl.BlockSpec((R, CW), lambda i: (i, 0)),
            pl.BlockSpec((2, HALF, D), lambda i: (0, 0, 0)),
            pl.BlockSpec((2, HALF, D), lambda i: (0, 0, 0)),
            pl.BlockSpec((1, D), lambda i: (0, 0)),
        ],
        out_specs=[row_spec, row_spec],
        out_shape=[
            jax.ShapeDtypeStruct((N, HALF), jnp.float32),
            jax.ShapeDtypeStruct((N, HALF), jnp.float32),
        ],
    )(sL, sR, hL, hR, cnt, Wl, Wr, b)


def _tc_final_body(sL, sR, hL, hR, cnt, Wl, Wr, b, Wlin, blin, out,
                   m_scr):
    i = pl.program_id(0)
    inv = 1.0 / jnp.maximum(cnt[:, 0:1], 1.0)
    acc = (jnp.dot(sL[...] * inv, Wl[0], precision="highest")
           + jnp.dot(sR[...] * inv, Wl[1], precision="highest")
           + jnp.dot(hL[...], Wr[0], precision="highest")
           + jnp.dot(hR[...], Wr[1], precision="highest")
           + b[...])
    bmax = jnp.max(acc.reshape(R // 8, 8, D), axis=0)

    @pl.when(i == 0)
    def _():
        m_scr[...] = bmax

    @pl.when(i > 0)
    def _():
        m_scr[...] = jnp.maximum(m_scr[...], bmax)

    @pl.when(i == pl.num_programs(0) - 1)
    def _():
        pooled = jnp.max(m_scr[...], axis=0, keepdims=True)
        p8 = jnp.broadcast_to(pooled, (8, D))
        res = jnp.dot(p8, Wlin[...], precision="highest") + blin[...]
        out[...] = res[0:1]


def _tc_final(sL, sR, hL, hR, cnt, Wl, Wr, b, Wlin, blin):
    grid = (N // R,)
    row_spec = pl.BlockSpec((R, HALF), lambda i: (i, 0))
    return pl.pallas_call(
        _tc_final_body,
        grid=grid,
        in_specs=[
            row_spec, row_spec, row_spec, row_spec,
            pl.BlockSpec((R, CW), lambda i: (i, 0)),
            pl.BlockSpec((2, HALF, D), lambda i: (0, 0, 0)),
            pl.BlockSpec((2, HALF, D), lambda i: (0, 0, 0)),
            pl.BlockSpec((1, D), lambda i: (0, 0)),
            pl.BlockSpec((D, OUT), lambda i: (0, 0)),
            pl.BlockSpec((1, OUT), lambda i: (0, 0)),
        ],
        out_specs=pl.BlockSpec((1, OUT), lambda i: (0, 0)),
        out_shape=jax.ShapeDtypeStruct((1, OUT), jnp.float32),
        scratch_shapes=[pltpu.VMEM((8, D), jnp.float32)],
    )(sL, sR, hL, hR, cnt, Wl, Wr, b, Wlin, blin)


def kernel(x, edge_index, W1l, b1l, W1r, W2l, b2l, W2r, W3l, b3l, W3r,
           Wlin, blin):
    xL = x[:, :HALF]
    xR = x[:, HALF:]
    srcr = edge_index[0].reshape(NUM_TILES, NB, KB)
    dstr = edge_index[1].reshape(NUM_TILES, NB, KB)

    def wsplit(W):
        return W.reshape(2, HALF, W.shape[1])

    ones_hbm = jnp.ones((KB, CW), jnp.float32)
    zrow = jnp.zeros((ROWS_PER_TILE, HALF), jnp.float32)
    zcnt = jnp.zeros((ROWS_PER_TILE, CW), jnp.float32)

    s1L, s1R, cnt = _sc_seg_count(xL, xR, srcr, dstr, ones_hbm, zrow, zcnt)
    h1L, h1R = _tc_layer(s1L, s1R, xL, xR, cnt,
                         wsplit(W1l), wsplit(W1r), b1l.reshape(1, D))
    s2L, s2R = _sc_seg(h1L, h1R, srcr, dstr, zrow)
    h2L, h2R = _tc_layer(s2L, s2R, h1L, h1R, cnt,
                         wsplit(W2l), wsplit(W2r), b2l.reshape(1, D))
    s3L, s3R = _sc_seg(h2L, h2R, srcr, dstr, zrow)
    return _tc_final(s3L, s3R, h2L, h2R, cnt,
                     wsplit(W3l), wsplit(W3r), b3l.reshape(1, D),
                     Wlin, blin.reshape(1, OUT))


# speedup vs baseline: 4.4816x; 4.4816x over previous
import functools

import jax
import jax.numpy as jnp
from jax import lax
from jax.experimental import pallas as pl
from jax.experimental.pallas import tpu as pltpu
from jax.experimental.pallas import tpu_sc as plsc

N = 10000
E = 160000
D = 256
HALF = 128
OUT = 128

NUM_TILES = 16
EDGES_PER_TILE = E // NUM_TILES
KB = 200
NB = EDGES_PER_TILE // KB
ROWS_PER_TILE = 624
TAIL0 = ROWS_PER_TILE * NUM_TILES
TAILN = N - TAIL0


def _sc_segment_sum(count: bool):
    mesh = plsc.VectorSubcoreMesh(core_axis_name="c", subcore_axis_name="s")
    out_type = [
        jax.ShapeDtypeStruct((N, HALF), jnp.float32),
        jax.ShapeDtypeStruct((N, HALF), jnp.float32),
    ]
    if count:
        out_type.append(jax.ShapeDtypeStruct((N,), jnp.float32))

    scratch = [
        pltpu.VMEM_SHARED((N, HALF), jnp.float32),
        pltpu.VMEM((KB,), jnp.int32),
        pltpu.VMEM((KB,), jnp.int32),
        pltpu.VMEM((KB, HALF), jnp.float32),
        pltpu.SemaphoreType.DMA,
    ]
    if count:
        scratch.insert(1, pltpu.VMEM_SHARED((N,), jnp.float32))
        scratch.append(pltpu.VMEM((KB,), jnp.float32))
        scratch.append(pltpu.VMEM((ROWS_PER_TILE,), jnp.float32))

    @functools.partial(
        pl.kernel, mesh=mesh, out_type=tuple(out_type),
        scratch_types=scratch,
    )
    def k(*refs):
        if count:
            (tabL, tabR, srcr, dstr, ones_hbm, zrow, zcnt,
             sumL, sumR, cnt_out,
             acc, cacc, idx_s, idx_d, rows, sem, ones_v, cb) = refs
        else:
            (tabL, tabR, srcr, dstr, zrow,
             sumL, sumR,
             acc, idx_s, idx_d, rows, sem) = refs

        c = lax.axis_index("c")
        s = lax.axis_index("s")
        row0 = s * ROWS_PER_TILE
        last = s == NUM_TILES - 1

        def stripe(src, dst):
            pltpu.sync_copy(src.at[pl.ds(row0, ROWS_PER_TILE)],
                            dst.at[pl.ds(row0, ROWS_PER_TILE)])

            @pl.when(last)
            def _():
                pltpu.sync_copy(src.at[pl.ds(TAIL0, TAILN)],
                                dst.at[pl.ds(TAIL0, TAILN)])

        def zero_stripe(zsrc, dst):
            pltpu.sync_copy(zsrc, dst.at[pl.ds(row0, ROWS_PER_TILE)])

            @pl.when(last)
            def _():
                pltpu.sync_copy(zsrc.at[pl.ds(0, TAILN)],
                                dst.at[pl.ds(TAIL0, TAILN)])

        zero_stripe(zrow, acc)

        def run(table, out, with_count):
            if with_count:
                pltpu.sync_copy(zcnt, cb)
                pltpu.sync_copy(cb, cacc.at[pl.ds(row0, ROWS_PER_TILE)])

                @pl.when(last)
                def _():
                    pltpu.sync_copy(cb.at[pl.ds(0, TAILN)],
                                    cacc.at[pl.ds(TAIL0, TAILN)])

                pltpu.sync_copy(ones_hbm, ones_v)
            plsc.subcore_barrier()
            base = s * EDGES_PER_TILE
            for j in range(NB):
                pltpu.sync_copy(srcr.at[pl.ds(base + j * KB, KB)], idx_s)
                pltpu.sync_copy(dstr.at[pl.ds(base + j * KB, KB)], idx_d)
                pltpu.async_copy(table.at[idx_s], rows, sem).wait()
                pltpu.sync_copy(rows, acc.at[idx_d], add=True)
                if with_count:
                    pltpu.sync_copy(ones_v, cacc.at[idx_d], add=True)
            plsc.subcore_barrier()
            stripe(acc, out)
            if with_count:
                pltpu.sync_copy(cacc.at[pl.ds(row0, ROWS_PER_TILE)], cb)
                pltpu.sync_copy(cb, cnt_out.at[pl.ds(row0, ROWS_PER_TILE)])

                @pl.when(last)
                def _():
                    pltpu.sync_copy(cacc.at[pl.ds(TAIL0, TAILN)],
                                    cb.at[pl.ds(0, TAILN)])
                    pltpu.sync_copy(cb.at[pl.ds(0, TAILN)],
                                    cnt_out.at[pl.ds(TAIL0, TAILN)])

        @pl.when(c == 0)
        def _():
            run(tabL, sumL, count)

        @pl.when(c == 1)
        def _():
            run(tabR, sumR, False)

    return k


_sc_seg_count = _sc_segment_sum(True)
_sc_seg = _sc_segment_sum(False)

R = 1000


def _tc_layer_body(sL, sR, hL, hR, cnt, Wl, Wr, b, oL, oR):
    inv = 1.0 / jnp.maximum(cnt[:, 0:1], 1.0)
    acc = (jnp.dot(sL[...] * inv, Wl[0], precision="highest")
           + jnp.dot(sR[...] * inv, Wl[1], precision="highest")
           + jnp.dot(hL[...], Wr[0], precision="highest")
           + jnp.dot(hR[...], Wr[1], precision="highest")
           + b[...])
    acc = jnp.maximum(acc, 0.0)
    oL[...] = acc[:, :HALF]
    oR[...] = acc[:, HALF:]


def _tc_layer(sL, sR, hL, hR, cnt, Wl, Wr, b):
    grid = (N // R,)
    row_spec = pl.BlockSpec((R, HALF), lambda i: (i, 0))
    return pl.pallas_call(
        _tc_layer_body,
        grid=grid,
        in_specs=[
            row_spec, row_spec, row_spec, row_spec,
            pl.BlockSpec((R, 1), lambda i: (i, 0)),
            pl.BlockSpec((2, HALF, D), lambda i: (0, 0, 0)),
            pl.BlockSpec((2, HALF, D), lambda i: (0, 0, 0)),
            pl.BlockSpec((1, D), lambda i: (0, 0)),
        ],
        out_specs=[row_spec, row_spec],
        out_shape=[
            jax.ShapeDtypeStruct((N, HALF), jnp.float32),
            jax.ShapeDtypeStruct((N, HALF), jnp.float32),
        ],
    )(sL, sR, hL, hR, cnt, Wl, Wr, b)


def _tc_final_body(sL, sR, hL, hR, cnt, Wl, Wr, b, Wlin, blin, out,
                   m_scr):
    i = pl.program_id(0)
    inv = 1.0 / jnp.maximum(cnt[:, 0:1], 1.0)
    acc = (jnp.dot(sL[...] * inv, Wl[0], precision="highest")
           + jnp.dot(sR[...] * inv, Wl[1], precision="highest")
           + jnp.dot(hL[...], Wr[0], precision="highest")
           + jnp.dot(hR[...], Wr[1], precision="highest")
           + b[...])
    bmax = jnp.max(acc.reshape(R // 8, 8, D), axis=0)

    @pl.when(i == 0)
    def _():
        m_scr[...] = bmax

    @pl.when(i > 0)
    def _():
        m_scr[...] = jnp.maximum(m_scr[...], bmax)

    @pl.when(i == pl.num_programs(0) - 1)
    def _():
        pooled = jnp.max(m_scr[...], axis=0, keepdims=True)
        p8 = jnp.broadcast_to(pooled, (8, D))
        res = jnp.dot(p8, Wlin[...], precision="highest") + blin[...]
        out[...] = res[0:1]


def _tc_final(sL, sR, hL, hR, cnt, Wl, Wr, b, Wlin, blin):
    grid = (N // R,)
    row_spec = pl.BlockSpec((R, HALF), lambda i: (i, 0))
    return pl.pallas_call(
        _tc_final_body,
        grid=grid,
        in_specs=[
            row_spec, row_spec, row_spec, row_spec,
            pl.BlockSpec((R, 1), lambda i: (i, 0)),
            pl.BlockSpec((2, HALF, D), lambda i: (0, 0, 0)),
            pl.BlockSpec((2, HALF, D), lambda i: (0, 0, 0)),
            pl.BlockSpec((1, D), lambda i: (0, 0)),
            pl.BlockSpec((D, OUT), lambda i: (0, 0)),
            pl.BlockSpec((1, OUT), lambda i: (0, 0)),
        ],
        out_specs=pl.BlockSpec((1, OUT), lambda i: (0, 0)),
        out_shape=jax.ShapeDtypeStruct((1, OUT), jnp.float32),
        scratch_shapes=[pltpu.VMEM((8, D), jnp.float32)],
    )(sL, sR, hL, hR, cnt, Wl, Wr, b, Wlin, blin)


def kernel(x, edge_index, W1l, b1l, W1r, W2l, b2l, W2r, W3l, b3l, W3r,
           Wlin, blin):
    xL = x[:, :HALF]
    xR = x[:, HALF:]
    srcr = edge_index[0]
    dstr = edge_index[1]

    def wsplit(W):
        return W.reshape(2, HALF, W.shape[1])

    ones_hbm = jnp.ones((KB,), jnp.float32)
    zrow = jnp.zeros((ROWS_PER_TILE, HALF), jnp.float32)
    zcnt = jnp.zeros((ROWS_PER_TILE,), jnp.float32)

    s1L, s1R, cnt = _sc_seg_count(xL, xR, srcr, dstr, ones_hbm, zrow, zcnt)
    cnt = cnt.reshape(N, 1)
    h1L, h1R = _tc_layer(s1L, s1R, xL, xR, cnt,
                         wsplit(W1l), wsplit(W1r), b1l.reshape(1, D))
    s2L, s2R = _sc_seg(h1L, h1R, srcr, dstr, zrow)
    h2L, h2R = _tc_layer(s2L, s2R, h1L, h1R, cnt,
                         wsplit(W2l), wsplit(W2r), b2l.reshape(1, D))
    s3L, s3R = _sc_seg(h2L, h2R, srcr, dstr, zrow)
    return _tc_final(s3L, s3R, h2L, h2R, cnt,
                     wsplit(W3l), wsplit(W3r), b3l.reshape(1, D),
                     Wlin, blin.reshape(1, OUT))
